# baseline (speedup 1.0000x reference)
import functools

import jax
import jax.numpy as jnp
from jax import lax
from jax.experimental import pallas as pl
from jax.experimental.pallas import tpu as pltpu
from jax.experimental.pallas import tpu_sc as plsc

N_NEURONS = 4096
N_POST = 4096
MAX_DELAY = 16
L = 16
NC = 2
NS = 16
NW = NC * NS
ROWS_PER_W = N_NEURONS // NW
VECS_PER_ROW = N_POST // L
B = 4
NBLK = ROWS_PER_W // B
NBUF = 3


def _sc_body(delay_hbm, spikes_hbm, buffer_hbm, out_hbm, bt, packed,
             din0, din1, din2, dout0, dout1, dout2,
             sin0, sin1, sin2, sout0, sout1, sout2):
    wid = lax.axis_index("s") * NC + lax.axis_index("c")
    i0 = wid * ROWS_PER_W
    din = (din0, din1, din2)
    dout = (dout0, dout1, dout2)
    sin = (sin0, sin1, sin2)
    sout = (sout0, sout1, sout2)

    for d in range(MAX_DELAY):
        pltpu.sync_copy(buffer_hbm.at[d, pl.ds(i0, ROWS_PER_W)],
                        bt.at[pl.ds(d * ROWS_PER_W, ROWS_PER_W)])
    pltpu.sync_copy(spikes_hbm.at[pl.ds(i0, ROWS_PER_W)],
                    bt.at[pl.ds(0, ROWS_PER_W)])

    def pack_chunk(c, carry):
        acc = jnp.zeros((L,), jnp.float32)
        for d in range(1, MAX_DELAY + 1):
            rr = (17 - d) & 15
            v = bt[pl.ds(rr * ROWS_PER_W + c * L, L)]
            acc = acc + v * jnp.float32(1 << d)
        packed[pl.ds(c * L, L)] = acc.astype(jnp.int32)
        return carry

    lax.fori_loop(0, ROWS_PER_W // L, pack_chunk, 0)

    def in_copy(blk, buf):
        return pltpu.make_async_copy(
            delay_hbm.at[pl.ds(i0 + blk * B, B)], din[buf], sin[buf])

    def out_copy(blk, buf):
        return pltpu.make_async_copy(
            dout[buf], out_hbm.at[pl.ds(i0 + blk * B, B)], sout[buf])

    in_copy(0, 0).start()
    in_copy(1, 1).start()
    pv16 = None
    for blk in range(NBLK):
        buf = blk % NBUF
        if blk + 2 < NBLK:
            in_copy(blk + 2, (blk + 2) % NBUF).start()
        in_copy(blk, buf).wait()
        if blk >= NBUF:
            out_copy(blk - NBUF, buf).wait()
        if blk % 4 == 0:
            pv16 = packed[pl.ds((blk // 4) * L, L)]
        for k in range(B):
            p = jnp.broadcast_to(pv16[(blk % 4) * B + k], (L,))

            def vec_body(v, cc, _p=p, _k=k, _buf=buf):
                dvec = din[_buf][_k, pl.ds(v * L, L)]
                dout[_buf][_k, pl.ds(v * L, L)] = (
                    (_p >> dvec) & 1).astype(jnp.float32)
                return cc

            lax.fori_loop(0, VECS_PER_ROW, vec_body, 0, unroll=4)
        out_copy(blk, buf).start()
    for blk in range(NBLK - NBUF, NBLK):
        out_copy(blk, blk % NBUF).wait()


@functools.lru_cache(maxsize=1)
def _build():
    return pl.kernel(
        _sc_body,
        out_type=jax.ShapeDtypeStruct((N_NEURONS, N_POST), jnp.float32),
        mesh=plsc.VectorSubcoreMesh(
            core_axis_name="c", subcore_axis_name="s", num_cores=NC,
            num_subcores=NS),
        scratch_types=[
            pltpu.VMEM((MAX_DELAY * ROWS_PER_W,), jnp.float32),
            pltpu.VMEM((ROWS_PER_W,), jnp.int32),
            pltpu.VMEM((B, N_POST), jnp.int32),
            pltpu.VMEM((B, N_POST), jnp.int32),
            pltpu.VMEM((B, N_POST), jnp.int32),
            pltpu.VMEM((B, N_POST), jnp.float32),
            pltpu.VMEM((B, N_POST), jnp.float32),
            pltpu.VMEM((B, N_POST), jnp.float32),
            pltpu.SemaphoreType.DMA,
            pltpu.SemaphoreType.DMA,
            pltpu.SemaphoreType.DMA,
            pltpu.SemaphoreType.DMA,
            pltpu.SemaphoreType.DMA,
            pltpu.SemaphoreType.DMA,
        ],
    )


def kernel(spikes, delay_matrix, buffer):
    return _build()(delay_matrix, spikes, buffer)


# speedup vs baseline: 1.8987x; 1.8987x over previous
import functools

import jax
import jax.numpy as jnp
from jax import lax
from jax.experimental import pallas as pl
from jax.experimental.pallas import tpu as pltpu
from jax.experimental.pallas import tpu_sc as plsc

N_NEURONS = 4096
N_POST = 4096
MAX_DELAY = 16
L = 16
NC = 2
NS = 16
NW = NC * NS
ROWS_PER_W = N_NEURONS // NW
VECS_PER_ROW = N_POST // L
B = 4
NBLK = ROWS_PER_W // B
NBUF = 3


def _sc_body(delay_hbm, spikes_hbm, buffer_hbm, out_hbm, bt, packed,
             din0, din1, din2, dout0, dout1, dout2,
             sin0, sin1, sin2, sout0, sout1, sout2):
    wid = lax.axis_index("s") * NC + lax.axis_index("c")
    i0 = wid * ROWS_PER_W
    din = (din0, din1, din2)
    dout = (dout0, dout1, dout2)
    sin = (sin0, sin1, sin2)
    sout = (sout0, sout1, sout2)

    for d in range(MAX_DELAY):
        pltpu.sync_copy(buffer_hbm.at[d, pl.ds(i0, ROWS_PER_W)],
                        bt.at[pl.ds(d * ROWS_PER_W, ROWS_PER_W)])
    pltpu.sync_copy(spikes_hbm.at[pl.ds(i0, ROWS_PER_W)],
                    bt.at[pl.ds(0, ROWS_PER_W)])

    def pack_chunk(c, carry):
        acc = jnp.zeros((L,), jnp.float32)
        for d in range(1, MAX_DELAY + 1):
            rr = (17 - d) & 15
            v = bt[pl.ds(rr * ROWS_PER_W + c * L, L)]
            acc = acc + v * jnp.float32(1 << d)
        packed[pl.ds(c * L, L)] = acc.astype(jnp.int32)
        return carry

    lax.fori_loop(0, ROWS_PER_W // L, pack_chunk, 0)

    def in_copy(blk, buf):
        return pltpu.make_async_copy(
            delay_hbm.at[pl.ds(i0 + blk * B, B)], din[buf], sin[buf])

    def out_copy(blk, buf):
        return pltpu.make_async_copy(
            dout[buf], out_hbm.at[pl.ds(i0 + blk * B, B)], sout[buf])

    in_copy(0, 0).start()
    in_copy(1, 1).start()
    pv16 = None
    for blk in range(NBLK):
        buf = blk % NBUF
        if blk + 2 < NBLK:
            in_copy(blk + 2, (blk + 2) % NBUF).start()
        in_copy(blk, buf).wait()
        if blk >= NBUF:
            out_copy(blk - NBUF, buf).wait()
        if blk % 4 == 0:
            pv16 = packed[pl.ds((blk // 4) * L, L)]
        for k in range(B):
            p = jnp.broadcast_to(pv16[(blk % 4) * B + k], (L,))

            @plsc.parallel_loop(0, VECS_PER_ROW, unroll=4)
            def vec_body(v, _p=p, _k=k, _buf=buf):
                dvec = din[_buf][_k, pl.ds(v * L, L)]
                dout[_buf][_k, pl.ds(v * L, L)] = (
                    (_p >> dvec) & 1).astype(jnp.float32)
        out_copy(blk, buf).start()
    for blk in range(NBLK - NBUF, NBLK):
        out_copy(blk, blk % NBUF).wait()


@functools.lru_cache(maxsize=1)
def _build():
    return pl.kernel(
        _sc_body,
        out_type=jax.ShapeDtypeStruct((N_NEURONS, N_POST), jnp.float32),
        mesh=plsc.VectorSubcoreMesh(
            core_axis_name="c", subcore_axis_name="s", num_cores=NC,
            num_subcores=NS),
        scratch_types=[
            pltpu.VMEM((MAX_DELAY * ROWS_PER_W,), jnp.float32),
            pltpu.VMEM((ROWS_PER_W,), jnp.int32),
            pltpu.VMEM((B, N_POST), jnp.int32),
            pltpu.VMEM((B, N_POST), jnp.int32),
            pltpu.VMEM((B, N_POST), jnp.int32),
            pltpu.VMEM((B, N_POST), jnp.float32),
            pltpu.VMEM((B, N_POST), jnp.float32),
            pltpu.VMEM((B, N_POST), jnp.float32),
            pltpu.SemaphoreType.DMA,
            pltpu.SemaphoreType.DMA,
            pltpu.SemaphoreType.DMA,
            pltpu.SemaphoreType.DMA,
            pltpu.SemaphoreType.DMA,
            pltpu.SemaphoreType.DMA,
        ],
    )


def kernel(spikes, delay_matrix, buffer):
    return _build()(delay_matrix, spikes, buffer)


# speedup vs baseline: 1.9103x; 1.0061x over previous
import functools

import jax
import jax.numpy as jnp
from jax import lax
from jax.experimental import pallas as pl
from jax.experimental.pallas import tpu as pltpu
from jax.experimental.pallas import tpu_sc as plsc

N_NEURONS = 4096
N_POST = 4096
MAX_DELAY = 16
L = 16
NC = 2
NS = 16
NW = NC * NS
ROWS_PER_W = N_NEURONS // NW
VECS_PER_ROW = N_POST // L
B = 4
NBLK = ROWS_PER_W // B
NBUF = 3


def _sc_body(delay_hbm, spikes_hbm, buffer_hbm, out_hbm, bt, packed,
             din0, din1, din2, dout0, dout1, dout2,
             sin0, sin1, sin2, sout0, sout1, sout2):
    wid = lax.axis_index("s") * NC + lax.axis_index("c")
    i0 = wid * ROWS_PER_W
    din = (din0, din1, din2)
    dout = (dout0, dout1, dout2)
    sin = (sin0, sin1, sin2)
    sout = (sout0, sout1, sout2)

    for d in range(MAX_DELAY):
        pltpu.sync_copy(buffer_hbm.at[d, pl.ds(i0, ROWS_PER_W)],
                        bt.at[pl.ds(d * ROWS_PER_W, ROWS_PER_W)])
    pltpu.sync_copy(spikes_hbm.at[pl.ds(i0, ROWS_PER_W)],
                    bt.at[pl.ds(0, ROWS_PER_W)])

    def pack_chunk(c, carry):
        acc = jnp.zeros((L,), jnp.float32)
        for d in range(1, MAX_DELAY + 1):
            rr = (17 - d) & 15
            v = bt[pl.ds(rr * ROWS_PER_W + c * L, L)]
            acc = acc + v * jnp.float32(1 << d)
        packed[pl.ds(c * L, L)] = acc.astype(jnp.int32)
        return carry

    lax.fori_loop(0, ROWS_PER_W // L, pack_chunk, 0)

    def in_copy(blk, buf):
        return pltpu.make_async_copy(
            delay_hbm.at[pl.ds(i0 + blk * B, B)], din[buf], sin[buf])

    def out_copy(blk, buf):
        return pltpu.make_async_copy(
            dout[buf], out_hbm.at[pl.ds(i0 + blk * B, B)], sout[buf])

    in_copy(0, 0).start()
    in_copy(1, 1).start()
    pv16 = None
    for blk in range(NBLK):
        buf = blk % NBUF
        if blk + 2 < NBLK:
            in_copy(blk + 2, (blk + 2) % NBUF).start()
        in_copy(blk, buf).wait()
        if blk >= NBUF:
            out_copy(blk - NBUF, buf).wait()
        if blk % 4 == 0:
            pv16 = packed[pl.ds((blk // 4) * L, L)]
        for k in range(B):
            p = jnp.broadcast_to(pv16[(blk % 4) * B + k], (L,))

            @plsc.parallel_loop(0, VECS_PER_ROW, unroll=8)
            def vec_body(v, _p=p, _k=k, _buf=buf):
                dvec = din[_buf][_k, pl.ds(v * L, L)]
                dout[_buf][_k, pl.ds(v * L, L)] = (
                    (_p >> dvec) & 1).astype(jnp.float32)
        out_copy(blk, buf).start()
    for blk in range(NBLK - NBUF, NBLK):
        out_copy(blk, blk % NBUF).wait()


@functools.lru_cache(maxsize=1)
def _build():
    return pl.kernel(
        _sc_body,
        out_type=jax.ShapeDtypeStruct((N_NEURONS, N_POST), jnp.float32),
        mesh=plsc.VectorSubcoreMesh(
            core_axis_name="c", subcore_axis_name="s", num_cores=NC,
            num_subcores=NS),
        scratch_types=[
            pltpu.VMEM((MAX_DELAY * ROWS_PER_W,), jnp.float32),
            pltpu.VMEM((ROWS_PER_W,), jnp.int32),
            pltpu.VMEM((B, N_POST), jnp.int32),
            pltpu.VMEM((B, N_POST), jnp.int32),
            pltpu.VMEM((B, N_POST), jnp.int32),
            pltpu.VMEM((B, N_POST), jnp.float32),
            pltpu.VMEM((B, N_POST), jnp.float32),
            pltpu.VMEM((B, N_POST), jnp.float32),
            pltpu.SemaphoreType.DMA,
            pltpu.SemaphoreType.DMA,
            pltpu.SemaphoreType.DMA,
            pltpu.SemaphoreType.DMA,
            pltpu.SemaphoreType.DMA,
            pltpu.SemaphoreType.DMA,
        ],
    )


def kernel(spikes, delay_matrix, buffer):
    return _build()(delay_matrix, spikes, buffer)


# speedup vs baseline: 2.1276x; 1.1138x over previous
import functools

import jax
import jax.numpy as jnp
from jax import lax
from jax.experimental import pallas as pl
from jax.experimental.pallas import tpu as pltpu
from jax.experimental.pallas import tpu_sc as plsc

N_NEURONS = 4096
N_POST = 4096
MAX_DELAY = 16
L = 16
NC = 2
NS = 16
NW = NC * NS
ROWS_PER_W = N_NEURONS // NW
VECS_PER_ROW = N_POST // L
B = 4
NBLK = ROWS_PER_W // B
NBUF = 3


def _sc_body(delay_hbm, spikes_hbm, buffer_hbm, out_hbm, bt, spk, packed,
             din0, din1, din2, dout0, dout1, dout2,
             sin0, sin1, sin2, sout0, sout1, sout2):
    wid = lax.axis_index("s") * NC + lax.axis_index("c")
    i0 = wid * ROWS_PER_W
    din = (din0, din1, din2)
    dout = (dout0, dout1, dout2)
    sin = (sin0, sin1, sin2)
    sout = (sout0, sout1, sout2)

    def in_copy(blk, buf):
        return pltpu.make_async_copy(
            delay_hbm.at[pl.ds(i0 + blk * B, B)], din[buf], sin[buf])

    def out_copy(blk, buf):
        return pltpu.make_async_copy(
            dout[buf], out_hbm.at[pl.ds(i0 + blk * B, B)], sout[buf])

    in_copy(0, 0).start()
    in_copy(1, 1).start()

    pltpu.sync_copy(buffer_hbm.at[:, pl.ds(i0, ROWS_PER_W)], bt)
    pltpu.sync_copy(spikes_hbm.at[pl.ds(i0, ROWS_PER_W)], spk)

    def pack_chunk(c, carry):
        acc = jnp.zeros((L,), jnp.float32)
        for d in range(1, MAX_DELAY + 1):
            rr = (17 - d) & 15
            if rr == 0:
                v = spk[pl.ds(c * L, L)]
            else:
                v = bt[rr, pl.ds(c * L, L)]
            acc = acc + v * jnp.float32(1 << d)
        packed[pl.ds(c * L, L)] = acc.astype(jnp.int32)
        return carry

    lax.fori_loop(0, ROWS_PER_W // L, pack_chunk, 0)

    pv16 = None
    for blk in range(NBLK):
        buf = blk % NBUF
        if blk + 2 < NBLK:
            in_copy(blk + 2, (blk + 2) % NBUF).start()
        in_copy(blk, buf).wait()
        if blk >= NBUF:
            out_copy(blk - NBUF, buf).wait()
        if blk % 4 == 0:
            pv16 = packed[pl.ds((blk // 4) * L, L)]
        for k in range(B):
            p = jnp.broadcast_to(pv16[(blk % 4) * B + k], (L,))

            @plsc.parallel_loop(0, VECS_PER_ROW, unroll=8)
            def vec_body(v, _p=p, _k=k, _buf=buf):
                dvec = din[_buf][_k, pl.ds(v * L, L)]
                dout[_buf][_k, pl.ds(v * L, L)] = (
                    (_p >> dvec) & 1).astype(jnp.float32)
        out_copy(blk, buf).start()
    for blk in range(NBLK - NBUF, NBLK):
        out_copy(blk, blk % NBUF).wait()


@functools.lru_cache(maxsize=1)
def _build():
    return pl.kernel(
        _sc_body,
        out_type=jax.ShapeDtypeStruct((N_NEURONS, N_POST), jnp.float32),
        mesh=plsc.VectorSubcoreMesh(
            core_axis_name="c", subcore_axis_name="s", num_cores=NC,
            num_subcores=NS),
        scratch_types=[
            pltpu.VMEM((MAX_DELAY, ROWS_PER_W), jnp.float32),
            pltpu.VMEM((ROWS_PER_W,), jnp.float32),
            pltpu.VMEM((ROWS_PER_W,), jnp.int32),
            pltpu.VMEM((B, N_POST), jnp.int32),
            pltpu.VMEM((B, N_POST), jnp.int32),
            pltpu.VMEM((B, N_POST), jnp.int32),
            pltpu.VMEM((B, N_POST), jnp.float32),
            pltpu.VMEM((B, N_POST), jnp.float32),
            pltpu.VMEM((B, N_POST), jnp.float32),
            pltpu.SemaphoreType.DMA,
            pltpu.SemaphoreType.DMA,
            pltpu.SemaphoreType.DMA,
            pltpu.SemaphoreType.DMA,
            pltpu.SemaphoreType.DMA,
            pltpu.SemaphoreType.DMA,
        ],
    )


def kernel(spikes, delay_matrix, buffer):
    return _build()(delay_matrix, spikes, buffer)
